# Initial kernel scaffold; baseline (speedup 1.0000x reference)
import jax
import jax.numpy as jnp
from jax.experimental import pallas as pl


def kernel(x, edge_index, gcn1_W, gcn1_b, gcn2_W, gcn2_b, n1_g, n1_b, n2_g, n2_b, ne_g, ne_b, ei_W1, ei_b1, ei_W2, ei_b2, eu_W1, eu_b1, eu_W2, eu_b2, ew_W, ew_b):
    raise NotImplementedError("write your pallas kernel here")


# speedup vs baseline: 10.3840x; 10.3840x over previous
import functools

import jax
import jax.numpy as jnp
from jax import lax
from jax.experimental import pallas as pl
from jax.experimental.pallas import tpu as pltpu
from jax.experimental.pallas import tpu_sc as plsc

N = 10000
E = 320000
D = 128
ED = 64
HD = 128

NW = 32
EPW = E // NW

_MESH = dict(core_axis_name="c", subcore_axis_name="s")


def _wid():
    return lax.axis_index("s") * 2 + lax.axis_index("c")


def _sc_gather2(P, Q, row, col, Dd, CH):
    nsteps = EPW // CH

    @functools.partial(
        pl.kernel,
        out_type=jax.ShapeDtypeStruct((E, Dd), jnp.float32),
        mesh=plsc.VectorSubcoreMesh(**_MESH),
        scratch_types=[
            pltpu.VMEM((CH,), jnp.int32),
            pltpu.VMEM((CH, Dd), jnp.float32),
            pltpu.SemaphoreType.DMA,
        ],
        compiler_params=pltpu.CompilerParams(use_tc_tiling_on_sc=False),
    )
    def k(p_hbm, q_hbm, row_hbm, col_hbm, out_hbm, idx_v, buf, sem):
        base = _wid() * EPW

        def body(i, c):
            off = base + i * CH
            pltpu.sync_copy(row_hbm.at[pl.ds(off, CH)], idx_v)
            pltpu.async_copy(p_hbm.at[idx_v], buf, sem).wait()
            pltpu.sync_copy(col_hbm.at[pl.ds(off, CH)], idx_v)
            pltpu.async_copy(q_hbm.at[idx_v], buf, sem, add=True).wait()
            pltpu.sync_copy(buf, out_hbm.at[pl.ds(off, CH)])
            return c

        lax.fori_loop(0, nsteps, body, 0)

    return k(P, Q, row, col)


def _sc_degree(col, w):
    CH = 2000
    nsteps = EPW // CH

    @functools.partial(
        pl.kernel,
        out_type=jax.ShapeDtypeStruct((2, N), jnp.float32),
        mesh=plsc.VectorSubcoreMesh(**_MESH),
        scratch_types=[
            pltpu.VMEM((CH,), jnp.int32),
            pltpu.VMEM((CH,), jnp.float32),
            pltpu.VMEM_SHARED((N,), jnp.float32),
        ],
    )
    def k(col_hbm, w_hbm, out_hbm, idx_v, w_v, acc_sh):
        cid = lax.axis_index("c")
        sid = lax.axis_index("s")

        def z(i, c):
            w_v[pl.ds(i * 16, 16)] = jnp.zeros((16,), jnp.float32)
            return c

        lax.fori_loop(0, CH // 16, z, 0)

        nz = N // CH
        @pl.when(sid < nz)
        def _():
            pltpu.sync_copy(w_v, acc_sh.at[pl.ds(sid * CH, CH)])

        plsc.subcore_barrier()
        base = _wid() * EPW

        def body(i, c):
            off = base + i * CH
            pltpu.sync_copy(col_hbm.at[pl.ds(off, CH)], idx_v)
            pltpu.sync_copy(w_hbm.at[pl.ds(off, CH)], w_v)
            pltpu.sync_copy(w_v, acc_sh.at[idx_v], add=True)
            return c

        lax.fori_loop(0, nsteps, body, 0)
        plsc.subcore_barrier()

        @pl.when(sid == 0)
        def _():
            pltpu.sync_copy(acc_sh, out_hbm.at[cid])

    return k(col, w)


def _sc_aggregate(y, row, col, w):
    CH = 200
    nsteps = EPW // CH
    nz = N // CH

    @functools.partial(
        pl.kernel,
        out_type=jax.ShapeDtypeStruct((2, N, D), jnp.float32),
        mesh=plsc.VectorSubcoreMesh(**_MESH),
        scratch_types=[
            pltpu.VMEM((CH,), jnp.int32),
            pltpu.VMEM((CH + 16,), jnp.float32),
            pltpu.VMEM((CH, D), jnp.float32),
            pltpu.VMEM_SHARED((N, D), jnp.float32),
            pltpu.SemaphoreType.DMA,
        ],
    )
    def k(y_hbm, row_hbm, col_hbm, w_hbm, out_hbm, idx_v, w_v, buf, acc_sh, sem):
        cid = lax.axis_index("c")
        sid = lax.axis_index("s")

        def zrow(i, c):
            for kk in range(D // 16):
                buf[i, pl.ds(kk * 16, 16)] = jnp.zeros((16,), jnp.float32)
            return c

        lax.fori_loop(0, CH, zrow, 0)

        for jj in range((nz + 15) // 16):
            j = sid + 16 * jj

            @pl.when(j < nz)
            def _():
                pltpu.sync_copy(buf, acc_sh.at[pl.ds(j * CH, CH)])

        plsc.subcore_barrier()
        base = _wid() * EPW

        def body(i, c):
            off = base + i * CH
            pltpu.sync_copy(row_hbm.at[pl.ds(off, CH)], idx_v)
            pltpu.sync_copy(w_hbm.at[pl.ds(off, CH)], w_v.at[pl.ds(0, CH)])
            pltpu.async_copy(y_hbm.at[idx_v], buf, sem).wait()

            def srow(e, c2):
                we = w_v[pl.ds(e, 16)][0]
                for kk in range(D // 16):
                    buf[e, pl.ds(kk * 16, 16)] = buf[e, pl.ds(kk * 16, 16)] * we
                return c2

            lax.fori_loop(0, CH, srow, 0)
            pltpu.sync_copy(col_hbm.at[pl.ds(off, CH)], idx_v)
            pltpu.sync_copy(buf, acc_sh.at[idx_v], add=True)
            return c

        lax.fori_loop(0, nsteps, body, 0)
        plsc.subcore_barrier()

        for jj in range((nz + 15) // 16):
            j = sid + 16 * jj

            @pl.when(j < nz)
            def _():
                pltpu.sync_copy(acc_sh.at[pl.ds(j * CH, CH)],
                                out_hbm.at[cid, pl.ds(j * CH, CH)])

    return k(y, row, col, w)


_NB = 2000


def _tc_node1(x, W1a, W1b, b1):
    def body(x_ref, wa_ref, wb_ref, b1_ref, p_ref, q_ref):
        xb = x_ref[...]
        p_ref[...] = jnp.dot(xb, wa_ref[...], preferred_element_type=jnp.float32)
        q_ref[...] = jnp.dot(xb, wb_ref[...], preferred_element_type=jnp.float32) + b1_ref[...]

    return pl.pallas_call(
        body,
        grid=(N // _NB,),
        in_specs=[
            pl.BlockSpec((_NB, D), lambda i: (i, 0)),
            pl.BlockSpec((D, ED), lambda i: (0, 0)),
            pl.BlockSpec((D, ED), lambda i: (0, 0)),
            pl.BlockSpec((1, ED), lambda i: (0, 0)),
        ],
        out_specs=[
            pl.BlockSpec((_NB, ED), lambda i: (i, 0)),
            pl.BlockSpec((_NB, ED), lambda i: (i, 0)),
        ],
        out_shape=[jax.ShapeDtypeStruct((N, ED), jnp.float32)] * 2,
    )(x, W1a, W1b, b1.reshape(1, ED))


_EB = 4000
_NEB = E // _EB


def _tc_edge1(PQ, ei_W2, ei_b2, ew_W, ew_b):
    def body(pq_ref, w2_ref, b2_ref, eww_ref, ewb_ref, ef_ref, w1_ref):
        ef = jnp.dot(jnp.maximum(pq_ref[...], 0.0), w2_ref[...],
                     preferred_element_type=jnp.float32) + b2_ref[...]
        ef_ref[...] = ef
        logits = jnp.sum(ef * eww_ref[...], axis=1) + ewb_ref[0, 0]
        w1_ref[...] = jax.nn.sigmoid(logits).reshape(1, 1, _EB)

    ef, w1 = pl.pallas_call(
        body,
        grid=(_NEB,),
        in_specs=[
            pl.BlockSpec((_EB, ED), lambda i: (i, 0)),
            pl.BlockSpec((ED, ED), lambda i: (0, 0)),
            pl.BlockSpec((1, ED), lambda i: (0, 0)),
            pl.BlockSpec((1, ED), lambda i: (0, 0)),
            pl.BlockSpec((1, 1), lambda i: (0, 0)),
        ],
        out_specs=[
            pl.BlockSpec((_EB, ED), lambda i: (i, 0)),
            pl.BlockSpec((1, 1, _EB), lambda i: (i, 0, 0)),
        ],
        out_shape=[
            jax.ShapeDtypeStruct((E, ED), jnp.float32),
            jax.ShapeDtypeStruct((_NEB, 1, _EB), jnp.float32),
        ],
    )(PQ, ei_W2, ei_b2.reshape(1, ED), ew_W.reshape(1, ED), ew_b.reshape(1, 1))
    return ef, w1.reshape(E)


def _tc_mid(degp, x, gW):
    def body(degp_ref, x_ref, gw_ref, dinv_ref, y_ref):
        deg = 1.0 + degp_ref[0] + degp_ref[1]
        dinv = lax.rsqrt(deg)
        dinv_ref[...] = dinv
        xw = jnp.dot(x_ref[...], gw_ref[...], preferred_element_type=jnp.float32)
        y_ref[...] = xw * dinv

    return pl.pallas_call(
        body,
        grid=(N // _NB,),
        in_specs=[
            pl.BlockSpec((2, _NB, 1), lambda i: (0, i, 0)),
            pl.BlockSpec((_NB, D), lambda i: (i, 0)),
            pl.BlockSpec((D, D), lambda i: (0, 0)),
        ],
        out_specs=[
            pl.BlockSpec((_NB, 1), lambda i: (i, 0)),
            pl.BlockSpec((_NB, D), lambda i: (i, 0)),
        ],
        out_shape=[
            jax.ShapeDtypeStruct((N, 1), jnp.float32),
            jax.ShapeDtypeStruct((N, D), jnp.float32),
        ],
    )(degp.reshape(2, N, 1), x, gW)


def _gelu_ln(pre, g_ref, b_ref):
    h = 0.5 * pre * (1.0 + lax.erf(pre * (2.0 ** -0.5)))
    m = jnp.mean(h, axis=-1, keepdims=True)
    v = jnp.mean((h - m) ** 2, axis=-1, keepdims=True)
    return (h - m) * lax.rsqrt(v + 1e-5) * g_ref[...] + b_ref[...]


def _tc_out1(accp, y1, dinv1, gcn1_b, n1_g, n1_b, eu_A, eu_B, eu_b1, gcn2_W):
    def body(accp_ref, y_ref, dinv_ref, gb_ref, g_ref, b_ref, a_ref, bb_ref,
             eb1_ref, gw2_ref, r_ref, s_ref, xw2_ref):
        pre = dinv_ref[...] * (accp_ref[0] + accp_ref[1] + y_ref[...]) + gb_ref[...]
        out1 = _gelu_ln(pre, g_ref, b_ref)
        r_ref[...] = jnp.dot(out1, a_ref[...], preferred_element_type=jnp.float32)
        s_ref[...] = jnp.dot(out1, bb_ref[...], preferred_element_type=jnp.float32) + eb1_ref[...]
        xw2_ref[...] = jnp.dot(out1, gw2_ref[...], preferred_element_type=jnp.float32)

    return pl.pallas_call(
        body,
        grid=(N // _NB,),
        in_specs=[
            pl.BlockSpec((2, _NB, D), lambda i: (0, i, 0)),
            pl.BlockSpec((_NB, D), lambda i: (i, 0)),
            pl.BlockSpec((_NB, 1), lambda i: (i, 0)),
            pl.BlockSpec((1, D), lambda i: (0, 0)),
            pl.BlockSpec((1, D), lambda i: (0, 0)),
            pl.BlockSpec((1, D), lambda i: (0, 0)),
            pl.BlockSpec((D, D), lambda i: (0, 0)),
            pl.BlockSpec((D, D), lambda i: (0, 0)),
            pl.BlockSpec((1, D), lambda i: (0, 0)),
            pl.BlockSpec((D, D), lambda i: (0, 0)),
        ],
        out_specs=[
            pl.BlockSpec((_NB, D), lambda i: (i, 0)),
            pl.BlockSpec((_NB, D), lambda i: (i, 0)),
            pl.BlockSpec((_NB, D), lambda i: (i, 0)),
        ],
        out_shape=[jax.ShapeDtypeStruct((N, D), jnp.float32)] * 3,
    )(accp, y1, dinv1, gcn1_b.reshape(1, D), n1_g.reshape(1, D),
      n1_b.reshape(1, D), eu_A, eu_B, eu_b1.reshape(1, D), gcn2_W)


def _tc_edge2(RS, ef, eu_C, eu_W2, eu_b2, ne_g, ne_b, ew_W, ew_b):
    def body(rs_ref, ef_ref, c_ref, w2_ref, b2_ref, g_ref, b_ref, eww_ref,
             ewb_ref, w2o_ref):
        ef = ef_ref[...]
        h = jnp.maximum(
            rs_ref[...] + jnp.dot(ef, c_ref[...], preferred_element_type=jnp.float32),
            0.0)
        upd = jnp.dot(h, w2_ref[...], preferred_element_type=jnp.float32) + b2_ref[...]
        ef2 = ef + upd
        m = jnp.mean(ef2, axis=-1, keepdims=True)
        v = jnp.mean((ef2 - m) ** 2, axis=-1, keepdims=True)
        ef2 = (ef2 - m) * lax.rsqrt(v + 1e-5) * g_ref[...] + b_ref[...]
        logits = jnp.sum(ef2 * eww_ref[...], axis=1) + ewb_ref[0, 0]
        w2o_ref[...] = jax.nn.sigmoid(logits).reshape(1, 1, _EB)

    w2 = pl.pallas_call(
        body,
        grid=(_NEB,),
        in_specs=[
            pl.BlockSpec((_EB, D), lambda i: (i, 0)),
            pl.BlockSpec((_EB, ED), lambda i: (i, 0)),
            pl.BlockSpec((ED, D), lambda i: (0, 0)),
            pl.BlockSpec((D, ED), lambda i: (0, 0)),
            pl.BlockSpec((1, ED), lambda i: (0, 0)),
            pl.BlockSpec((1, ED), lambda i: (0, 0)),
            pl.BlockSpec((1, ED), lambda i: (0, 0)),
            pl.BlockSpec((1, ED), lambda i: (0, 0)),
            pl.BlockSpec((1, 1), lambda i: (0, 0)),
        ],
        out_specs=pl.BlockSpec((1, 1, _EB), lambda i: (i, 0, 0)),
        out_shape=jax.ShapeDtypeStruct((_NEB, 1, _EB), jnp.float32),
    )(RS, ef, eu_C, eu_W2, eu_b2.reshape(1, ED), ne_g.reshape(1, ED),
      ne_b.reshape(1, ED), ew_W.reshape(1, ED), ew_b.reshape(1, 1))
    return w2.reshape(E)


def _tc_out2(accp, y2, dinv2, gcn2_b, n2_g, n2_b):
    def body(accp_ref, y_ref, dinv_ref, gb_ref, g_ref, b_ref, o_ref):
        pre = dinv_ref[...] * (accp_ref[0] + accp_ref[1] + y_ref[...]) + gb_ref[...]
        o_ref[...] = _gelu_ln(pre, g_ref, b_ref)

    return pl.pallas_call(
        body,
        grid=(N // _NB,),
        in_specs=[
            pl.BlockSpec((2, _NB, D), lambda i: (0, i, 0)),
            pl.BlockSpec((_NB, D), lambda i: (i, 0)),
            pl.BlockSpec((_NB, 1), lambda i: (i, 0)),
            pl.BlockSpec((1, D), lambda i: (0, 0)),
            pl.BlockSpec((1, D), lambda i: (0, 0)),
            pl.BlockSpec((1, D), lambda i: (0, 0)),
        ],
        out_specs=pl.BlockSpec((_NB, D), lambda i: (i, 0)),
        out_shape=jax.ShapeDtypeStruct((N, D), jnp.float32),
    )(accp, y2, dinv2, gcn2_b.reshape(1, D), n2_g.reshape(1, D),
      n2_b.reshape(1, D))


def kernel(x, edge_index, gcn1_W, gcn1_b, gcn2_W, gcn2_b, n1_g, n1_b, n2_g,
           n2_b, ne_g, ne_b, ei_W1, ei_b1, ei_W2, ei_b2, eu_W1, eu_b1, eu_W2,
           eu_b2, ew_W, ew_b):
    row = edge_index[0]
    col = edge_index[1]

    P, Q = _tc_node1(x, ei_W1[:D], ei_W1[D:], ei_b1)
    PQ = _sc_gather2(P, Q, row, col, ED, 1000)
    ef, w1 = _tc_edge1(PQ, ei_W2, ei_b2, ew_W, ew_b)

    deg1p = _sc_degree(col, w1)
    dinv1, y1 = _tc_mid(deg1p, x, gcn1_W)
    acc1p = _sc_aggregate(y1, row, col, w1)
    R, S, xw2 = _tc_out1(acc1p, y1, dinv1, gcn1_b, n1_g, n1_b,
                         eu_W1[:D], eu_W1[D:2 * D], eu_b1, gcn2_W)

    RS = _sc_gather2(R, S, row, col, D, 400)
    w2 = _tc_edge2(RS, ef, eu_W1[2 * D:], eu_W2, eu_b2, ne_g, ne_b, ew_W, ew_b)

    deg2p = _sc_degree(col, w2)

    def _scale_body(degp_ref, xw_ref, dinv_ref, y_ref):
        deg = 1.0 + degp_ref[0] + degp_ref[1]
        dinv = lax.rsqrt(deg)
        dinv_ref[...] = dinv
        y_ref[...] = xw_ref[...] * dinv

    dinv2, y2 = pl.pallas_call(
        _scale_body,
        grid=(N // _NB,),
        in_specs=[
            pl.BlockSpec((2, _NB, 1), lambda i: (0, i, 0)),
            pl.BlockSpec((_NB, D), lambda i: (i, 0)),
        ],
        out_specs=[
            pl.BlockSpec((_NB, 1), lambda i: (i, 0)),
            pl.BlockSpec((_NB, D), lambda i: (i, 0)),
        ],
        out_shape=[
            jax.ShapeDtypeStruct((N, 1), jnp.float32),
            jax.ShapeDtypeStruct((N, D), jnp.float32),
        ],
    )(deg2p.reshape(2, N, 1), xw2)

    acc2p = _sc_aggregate(y2, row, col, w2)
    return _tc_out2(acc2p, y2, dinv2, gcn2_b, n2_g, n2_b)


# speedup vs baseline: 10.5432x; 1.0153x over previous
import functools

import jax
import jax.numpy as jnp
from jax import lax
from jax.experimental import pallas as pl
from jax.experimental.pallas import tpu as pltpu
from jax.experimental.pallas import tpu_sc as plsc

N = 10000
E = 320000
D = 128
ED = 64
HD = 128

NW = 32
EPW = E // NW

_MESH = dict(core_axis_name="c", subcore_axis_name="s")


def _wid():
    return lax.axis_index("s") * 2 + lax.axis_index("c")


def _sc_gather2(P, Q, row, col, Dd, CH, pair_out=False):
    nsteps = EPW // CH
    oshape = (E // 2, 2 * Dd) if pair_out else (E, Dd)

    @functools.partial(
        pl.kernel,
        out_type=jax.ShapeDtypeStruct(oshape, jnp.float32),
        mesh=plsc.VectorSubcoreMesh(**_MESH),
        scratch_types=[
            pltpu.VMEM((CH,), jnp.int32),
            pltpu.VMEM((CH, Dd), jnp.float32),
            pltpu.SemaphoreType.DMA,
        ],
        compiler_params=pltpu.CompilerParams(use_tc_tiling_on_sc=False),
    )
    def k(p_hbm, q_hbm, row_hbm, col_hbm, out_hbm, idx_v, buf, sem):
        wid = _wid()
        base = wid * EPW

        def body(i, c):
            off = base + i * CH
            pltpu.sync_copy(row_hbm.at[pl.ds(off, CH)], idx_v)
            pltpu.async_copy(p_hbm.at[idx_v], buf, sem).wait()
            pltpu.sync_copy(col_hbm.at[pl.ds(off, CH)], idx_v)
            pltpu.async_copy(q_hbm.at[idx_v], buf, sem, add=True).wait()
            if pair_out:
                @pl.when(wid < NW // 2)
                def _():
                    pltpu.sync_copy(buf, out_hbm.at[pl.ds(off, CH), pl.ds(0, Dd)])

                @pl.when(wid >= NW // 2)
                def _():
                    pltpu.sync_copy(
                        buf, out_hbm.at[pl.ds(off - E // 2, CH), pl.ds(Dd, Dd)])
            else:
                pltpu.sync_copy(buf, out_hbm.at[pl.ds(off, CH)])
            return c

        lax.fori_loop(0, nsteps, body, 0)

    return k(P, Q, row, col)


def _sc_degree(col, w):
    CH = 2000
    nsteps = EPW // CH

    @functools.partial(
        pl.kernel,
        out_type=jax.ShapeDtypeStruct((2, N), jnp.float32),
        mesh=plsc.VectorSubcoreMesh(**_MESH),
        scratch_types=[
            pltpu.VMEM((CH,), jnp.int32),
            pltpu.VMEM((CH,), jnp.float32),
            pltpu.VMEM_SHARED((N,), jnp.float32),
        ],
    )
    def k(col_hbm, w_hbm, out_hbm, idx_v, w_v, acc_sh):
        cid = lax.axis_index("c")
        sid = lax.axis_index("s")

        def z(i, c):
            w_v[pl.ds(i * 16, 16)] = jnp.zeros((16,), jnp.float32)
            return c

        lax.fori_loop(0, CH // 16, z, 0)

        nz = N // CH
        @pl.when(sid < nz)
        def _():
            pltpu.sync_copy(w_v, acc_sh.at[pl.ds(sid * CH, CH)])

        plsc.subcore_barrier()
        base = _wid() * EPW

        def body(i, c):
            off = base + i * CH
            pltpu.sync_copy(col_hbm.at[pl.ds(off, CH)], idx_v)
            pltpu.sync_copy(w_hbm.at[pl.ds(off, CH)], w_v)
            pltpu.sync_copy(w_v, acc_sh.at[idx_v], add=True)
            return c

        lax.fori_loop(0, nsteps, body, 0)
        plsc.subcore_barrier()

        @pl.when(sid == 0)
        def _():
            pltpu.sync_copy(acc_sh, out_hbm.at[cid])

    return k(col, w)


def _sc_aggregate(y, row, col, w):
    CH = 200
    nsteps = EPW // CH
    nz = N // CH

    @functools.partial(
        pl.kernel,
        out_type=jax.ShapeDtypeStruct((2, N, D), jnp.float32),
        mesh=plsc.VectorSubcoreMesh(**_MESH),
        scratch_types=[
            pltpu.VMEM((CH,), jnp.int32),
            pltpu.VMEM((CH + 16,), jnp.float32),
            pltpu.VMEM((CH, D), jnp.float32),
            pltpu.VMEM_SHARED((N, D), jnp.float32),
            pltpu.SemaphoreType.DMA,
        ],
    )
    def k(y_hbm, row_hbm, col_hbm, w_hbm, out_hbm, idx_v, w_v, buf, acc_sh, sem):
        cid = lax.axis_index("c")
        sid = lax.axis_index("s")

        def zrow(i, c):
            for kk in range(D // 16):
                buf[i, pl.ds(kk * 16, 16)] = jnp.zeros((16,), jnp.float32)
            return c

        lax.fori_loop(0, CH, zrow, 0)

        for jj in range((nz + 15) // 16):
            j = sid + 16 * jj

            @pl.when(j < nz)
            def _():
                pltpu.sync_copy(buf, acc_sh.at[pl.ds(j * CH, CH)])

        plsc.subcore_barrier()
        base = _wid() * EPW

        def body(i, c):
            off = base + i * CH
            pltpu.sync_copy(row_hbm.at[pl.ds(off, CH)], idx_v)
            pltpu.sync_copy(w_hbm.at[pl.ds(off, CH)], w_v.at[pl.ds(0, CH)])
            pltpu.async_copy(y_hbm.at[idx_v], buf, sem).wait()

            def srow(e, c2):
                we = w_v[pl.ds(e, 16)][0]
                for kk in range(D // 16):
                    buf[e, pl.ds(kk * 16, 16)] = buf[e, pl.ds(kk * 16, 16)] * we
                return c2

            lax.fori_loop(0, CH, srow, 0)
            pltpu.sync_copy(col_hbm.at[pl.ds(off, CH)], idx_v)
            pltpu.sync_copy(buf, acc_sh.at[idx_v], add=True)
            return c

        lax.fori_loop(0, nsteps, body, 0)
        plsc.subcore_barrier()

        for jj in range((nz + 15) // 16):
            j = sid + 16 * jj

            @pl.when(j < nz)
            def _():
                pltpu.sync_copy(acc_sh.at[pl.ds(j * CH, CH)],
                                out_hbm.at[cid, pl.ds(j * CH, CH)])

    return k(y, row, col, w)


_NB = 2000


def _tc_node1(x, W1a, W1b, b1):
    def body(x_ref, wa_ref, wb_ref, b1_ref, p_ref, q_ref):
        xb = x_ref[...]
        p_ref[...] = jnp.dot(xb, wa_ref[...], preferred_element_type=jnp.float32)
        q_ref[...] = jnp.dot(xb, wb_ref[...], preferred_element_type=jnp.float32) + b1_ref[...]

    return pl.pallas_call(
        body,
        grid=(N // _NB,),
        in_specs=[
            pl.BlockSpec((_NB, D), lambda i: (i, 0)),
            pl.BlockSpec((D, ED), lambda i: (0, 0)),
            pl.BlockSpec((D, ED), lambda i: (0, 0)),
            pl.BlockSpec((1, ED), lambda i: (0, 0)),
        ],
        out_specs=[
            pl.BlockSpec((_NB, ED), lambda i: (i, 0)),
            pl.BlockSpec((_NB, ED), lambda i: (i, 0)),
        ],
        out_shape=[jax.ShapeDtypeStruct((N, ED), jnp.float32)] * 2,
    )(x, W1a, W1b, b1.reshape(1, ED))


_EB = 4000
_NEB = E // _EB


_EB2 = _EB // 2


def _tc_edge1(PQ2, ei_W2, ei_b2, ew_W, ew_b):
    W2d = jnp.zeros((D, D), jnp.float32)
    W2d = W2d.at[:ED, :ED].set(ei_W2).at[ED:, ED:].set(ei_W2)
    b2p = jnp.concatenate([ei_b2, ei_b2]).reshape(1, D)
    ewp = jnp.concatenate([ew_W[:, 0], ew_W[:, 0]]).reshape(1, D)

    def body(pq_ref, w2d_ref, b2p_ref, ewp_ref, ewb_ref, ef_ref, we_ref, wo_ref):
        efp = jnp.dot(jnp.maximum(pq_ref[...], 0.0), w2d_ref[...],
                      preferred_element_type=jnp.float32) + b2p_ref[...]
        ef_ref[...] = efp
        lp = efp * ewp_ref[...]
        l1 = jnp.sum(lp[:, :ED], axis=1) + ewb_ref[0, 0]
        l2 = jnp.sum(lp[:, ED:], axis=1) + ewb_ref[0, 0]
        we_ref[...] = jax.nn.sigmoid(l1).reshape(1, 1, _EB2)
        wo_ref[...] = jax.nn.sigmoid(l2).reshape(1, 1, _EB2)

    ef, we, wo = pl.pallas_call(
        body,
        grid=(_NEB,),
        in_specs=[
            pl.BlockSpec((_EB2, D), lambda i: (i, 0)),
            pl.BlockSpec((D, D), lambda i: (0, 0)),
            pl.BlockSpec((1, D), lambda i: (0, 0)),
            pl.BlockSpec((1, D), lambda i: (0, 0)),
            pl.BlockSpec((1, 1), lambda i: (0, 0)),
        ],
        out_specs=[
            pl.BlockSpec((_EB2, D), lambda i: (i, 0)),
            pl.BlockSpec((1, 1, _EB2), lambda i: (i, 0, 0)),
            pl.BlockSpec((1, 1, _EB2), lambda i: (i, 0, 0)),
        ],
        out_shape=[
            jax.ShapeDtypeStruct((E // 2, D), jnp.float32),
            jax.ShapeDtypeStruct((_NEB, 1, _EB2), jnp.float32),
            jax.ShapeDtypeStruct((_NEB, 1, _EB2), jnp.float32),
        ],
    )(PQ2, W2d, b2p, ewp, ew_b.reshape(1, 1))
    w1 = jnp.concatenate([we.reshape(E // 2), wo.reshape(E // 2)])
    return ef, w1


def _tc_mid(degp, x, gW):
    def body(degp_ref, x_ref, gw_ref, dinv_ref, y_ref):
        deg = 1.0 + degp_ref[0] + degp_ref[1]
        dinv = lax.rsqrt(deg)
        dinv_ref[...] = dinv
        xw = jnp.dot(x_ref[...], gw_ref[...], preferred_element_type=jnp.float32)
        y_ref[...] = xw * dinv

    return pl.pallas_call(
        body,
        grid=(N // _NB,),
        in_specs=[
            pl.BlockSpec((2, _NB, 1), lambda i: (0, i, 0)),
            pl.BlockSpec((_NB, D), lambda i: (i, 0)),
            pl.BlockSpec((D, D), lambda i: (0, 0)),
        ],
        out_specs=[
            pl.BlockSpec((_NB, 1), lambda i: (i, 0)),
            pl.BlockSpec((_NB, D), lambda i: (i, 0)),
        ],
        out_shape=[
            jax.ShapeDtypeStruct((N, 1), jnp.float32),
            jax.ShapeDtypeStruct((N, D), jnp.float32),
        ],
    )(degp.reshape(2, N, 1), x, gW)


def _gelu_ln(pre, g_ref, b_ref):
    h = 0.5 * pre * (1.0 + lax.erf(pre * (2.0 ** -0.5)))
    m = jnp.mean(h, axis=-1, keepdims=True)
    v = jnp.mean((h - m) ** 2, axis=-1, keepdims=True)
    return (h - m) * lax.rsqrt(v + 1e-5) * g_ref[...] + b_ref[...]


def _tc_out1(accp, y1, dinv1, gcn1_b, n1_g, n1_b, eu_A, eu_B, eu_b1, gcn2_W):
    def body(accp_ref, y_ref, dinv_ref, gb_ref, g_ref, b_ref, a_ref, bb_ref,
             eb1_ref, gw2_ref, r_ref, s_ref, xw2_ref):
        pre = dinv_ref[...] * (accp_ref[0] + accp_ref[1] + y_ref[...]) + gb_ref[...]
        out1 = _gelu_ln(pre, g_ref, b_ref)
        r_ref[...] = jnp.dot(out1, a_ref[...], preferred_element_type=jnp.float32)
        s_ref[...] = jnp.dot(out1, bb_ref[...], preferred_element_type=jnp.float32) + eb1_ref[...]
        xw2_ref[...] = jnp.dot(out1, gw2_ref[...], preferred_element_type=jnp.float32)

    return pl.pallas_call(
        body,
        grid=(N // _NB,),
        in_specs=[
            pl.BlockSpec((2, _NB, D), lambda i: (0, i, 0)),
            pl.BlockSpec((_NB, D), lambda i: (i, 0)),
            pl.BlockSpec((_NB, 1), lambda i: (i, 0)),
            pl.BlockSpec((1, D), lambda i: (0, 0)),
            pl.BlockSpec((1, D), lambda i: (0, 0)),
            pl.BlockSpec((1, D), lambda i: (0, 0)),
            pl.BlockSpec((D, D), lambda i: (0, 0)),
            pl.BlockSpec((D, D), lambda i: (0, 0)),
            pl.BlockSpec((1, D), lambda i: (0, 0)),
            pl.BlockSpec((D, D), lambda i: (0, 0)),
        ],
        out_specs=[
            pl.BlockSpec((_NB, D), lambda i: (i, 0)),
            pl.BlockSpec((_NB, D), lambda i: (i, 0)),
            pl.BlockSpec((_NB, D), lambda i: (i, 0)),
        ],
        out_shape=[jax.ShapeDtypeStruct((N, D), jnp.float32)] * 3,
    )(accp, y1, dinv1, gcn1_b.reshape(1, D), n1_g.reshape(1, D),
      n1_b.reshape(1, D), eu_A, eu_B, eu_b1.reshape(1, D), gcn2_W)


def _tc_edge2(RS, ef, eu_C, eu_W2, eu_b2, ne_g, ne_b, ew_W, ew_b):
    gw = (ne_g * ew_W[:, 0]).reshape(1, ED)
    sgw = jnp.sum(ne_g * ew_W[:, 0]).reshape(1, 1)
    cb = (jnp.sum(ne_b * ew_W[:, 0]) + ew_b[0]).reshape(1, 1)
    noff = E // 2 // _EB2

    def _half(rs, ef, c_ref, w2_ref, b2_ref, gw_ref, sgw_ref, cb_ref):
        h = jnp.maximum(
            rs + jnp.dot(ef, c_ref[...], preferred_element_type=jnp.float32), 0.0)
        upd = jnp.dot(h, w2_ref[...], preferred_element_type=jnp.float32) + b2_ref[...]
        ef2 = ef + upd
        s0 = jnp.sum(ef2, axis=1, keepdims=True)
        s1 = jnp.sum(ef2 * gw_ref[...], axis=1, keepdims=True)
        s2 = jnp.sum(ef2 * ef2, axis=1, keepdims=True)
        m = s0 * (1.0 / ED)
        var = s2 * (1.0 / ED) - m * m
        logit = (s1 - m * sgw_ref[0, 0]) * lax.rsqrt(var + 1e-5) + cb_ref[0, 0]
        return jax.nn.sigmoid(logit).reshape(1, 1, _EB2)

    def body(rs1_ref, rs2_ref, ef_ref, c_ref, w2_ref, b2_ref, gw_ref, sgw_ref,
             cb_ref, w2e_ref, w2o_ref):
        efp = ef_ref[...]
        w2e_ref[...] = _half(rs1_ref[...], efp[:, :ED], c_ref, w2_ref, b2_ref,
                             gw_ref, sgw_ref, cb_ref)
        w2o_ref[...] = _half(rs2_ref[...], efp[:, ED:], c_ref, w2_ref, b2_ref,
                             gw_ref, sgw_ref, cb_ref)

    w2e, w2o = pl.pallas_call(
        body,
        grid=(_NEB,),
        in_specs=[
            pl.BlockSpec((_EB2, D), lambda i: (i, 0)),
            pl.BlockSpec((_EB2, D), lambda i: (i + noff, 0)),
            pl.BlockSpec((_EB2, D), lambda i: (i, 0)),
            pl.BlockSpec((ED, D), lambda i: (0, 0)),
            pl.BlockSpec((D, ED), lambda i: (0, 0)),
            pl.BlockSpec((1, ED), lambda i: (0, 0)),
            pl.BlockSpec((1, ED), lambda i: (0, 0)),
            pl.BlockSpec((1, 1), lambda i: (0, 0)),
            pl.BlockSpec((1, 1), lambda i: (0, 0)),
        ],
        out_specs=[
            pl.BlockSpec((1, 1, _EB2), lambda i: (i, 0, 0)),
            pl.BlockSpec((1, 1, _EB2), lambda i: (i, 0, 0)),
        ],
        out_shape=[
            jax.ShapeDtypeStruct((_NEB, 1, _EB2), jnp.float32),
            jax.ShapeDtypeStruct((_NEB, 1, _EB2), jnp.float32),
        ],
    )(RS, RS, ef, eu_C, eu_W2, eu_b2.reshape(1, ED), gw, sgw, cb)
    return jnp.concatenate([w2e.reshape(E // 2), w2o.reshape(E // 2)])


def _tc_out2(accp, y2, dinv2, gcn2_b, n2_g, n2_b):
    def body(accp_ref, y_ref, dinv_ref, gb_ref, g_ref, b_ref, o_ref):
        pre = dinv_ref[...] * (accp_ref[0] + accp_ref[1] + y_ref[...]) + gb_ref[...]
        o_ref[...] = _gelu_ln(pre, g_ref, b_ref)

    return pl.pallas_call(
        body,
        grid=(N // _NB,),
        in_specs=[
            pl.BlockSpec((2, _NB, D), lambda i: (0, i, 0)),
            pl.BlockSpec((_NB, D), lambda i: (i, 0)),
            pl.BlockSpec((_NB, 1), lambda i: (i, 0)),
            pl.BlockSpec((1, D), lambda i: (0, 0)),
            pl.BlockSpec((1, D), lambda i: (0, 0)),
            pl.BlockSpec((1, D), lambda i: (0, 0)),
        ],
        out_specs=pl.BlockSpec((_NB, D), lambda i: (i, 0)),
        out_shape=jax.ShapeDtypeStruct((N, D), jnp.float32),
    )(accp, y2, dinv2, gcn2_b.reshape(1, D), n2_g.reshape(1, D),
      n2_b.reshape(1, D))


def kernel(x, edge_index, gcn1_W, gcn1_b, gcn2_W, gcn2_b, n1_g, n1_b, n2_g,
           n2_b, ne_g, ne_b, ei_W1, ei_b1, ei_W2, ei_b2, eu_W1, eu_b1, eu_W2,
           eu_b2, ew_W, ew_b):
    row = edge_index[0]
    col = edge_index[1]

    P, Q = _tc_node1(x, ei_W1[:D], ei_W1[D:], ei_b1)
    PQ2 = _sc_gather2(P, Q, row, col, ED, 1000, pair_out=True)
    ef, w1 = _tc_edge1(PQ2, ei_W2, ei_b2, ew_W, ew_b)

    deg1p = _sc_degree(col, w1)
    dinv1, y1 = _tc_mid(deg1p, x, gcn1_W)
    acc1p = _sc_aggregate(y1, row, col, w1)
    R, S, xw2 = _tc_out1(acc1p, y1, dinv1, gcn1_b, n1_g, n1_b,
                         eu_W1[:D], eu_W1[D:2 * D], eu_b1, gcn2_W)

    RS = _sc_gather2(R, S, row, col, D, 400)
    w2 = _tc_edge2(RS, ef, eu_W1[2 * D:], eu_W2, eu_b2, ne_g, ne_b, ew_W, ew_b)

    deg2p = _sc_degree(col, w2)

    def _scale_body(degp_ref, xw_ref, dinv_ref, y_ref):
        deg = 1.0 + degp_ref[0] + degp_ref[1]
        dinv = lax.rsqrt(deg)
        dinv_ref[...] = dinv
        y_ref[...] = xw_ref[...] * dinv

    dinv2, y2 = pl.pallas_call(
        _scale_body,
        grid=(N // _NB,),
        in_specs=[
            pl.BlockSpec((2, _NB, 1), lambda i: (0, i, 0)),
            pl.BlockSpec((_NB, D), lambda i: (i, 0)),
        ],
        out_specs=[
            pl.BlockSpec((_NB, 1), lambda i: (i, 0)),
            pl.BlockSpec((_NB, D), lambda i: (i, 0)),
        ],
        out_shape=[
            jax.ShapeDtypeStruct((N, 1), jnp.float32),
            jax.ShapeDtypeStruct((N, D), jnp.float32),
        ],
    )(deg2p.reshape(2, N, 1), xw2)

    acc2p = _sc_aggregate(y2, row, col, w2)
    return _tc_out2(acc2p, y2, dinv2, gcn2_b, n2_g, n2_b)


# speedup vs baseline: 13.0559x; 1.2383x over previous
import functools

import jax
import jax.numpy as jnp
from jax import lax
from jax.experimental import pallas as pl
from jax.experimental.pallas import tpu as pltpu
from jax.experimental.pallas import tpu_sc as plsc

N = 10000
E = 320000
D = 128
ED = 64
HD = 128

NW = 32
EPW = E // NW

_MESH = dict(core_axis_name="c", subcore_axis_name="s")


def _wid():
    return lax.axis_index("s") * 2 + lax.axis_index("c")


def _sc_gather2(P, Q, row, col, Dd, CH, pair_out=False):
    nsteps = EPW // CH
    oshape = (E // 2, 2 * Dd) if pair_out else (E, Dd)

    @functools.partial(
        pl.kernel,
        out_type=jax.ShapeDtypeStruct(oshape, jnp.float32),
        mesh=plsc.VectorSubcoreMesh(**_MESH),
        scratch_types=[
            pltpu.VMEM((CH,), jnp.int32),
            pltpu.VMEM((CH, Dd), jnp.float32),
            pltpu.SemaphoreType.DMA,
        ],
        compiler_params=pltpu.CompilerParams(use_tc_tiling_on_sc=False),
    )
    def k(p_hbm, q_hbm, row_hbm, col_hbm, out_hbm, idx_v, buf, sem):
        wid = _wid()
        base = wid * EPW

        def body(i, c):
            off = base + i * CH
            pltpu.sync_copy(row_hbm.at[pl.ds(off, CH)], idx_v)
            pltpu.async_copy(p_hbm.at[idx_v], buf, sem).wait()
            pltpu.sync_copy(col_hbm.at[pl.ds(off, CH)], idx_v)
            pltpu.async_copy(q_hbm.at[idx_v], buf, sem, add=True).wait()
            if pair_out:
                @pl.when(wid < NW // 2)
                def _():
                    pltpu.sync_copy(buf, out_hbm.at[pl.ds(off, CH), pl.ds(0, Dd)])

                @pl.when(wid >= NW // 2)
                def _():
                    pltpu.sync_copy(
                        buf, out_hbm.at[pl.ds(off - E // 2, CH), pl.ds(Dd, Dd)])
            else:
                pltpu.sync_copy(buf, out_hbm.at[pl.ds(off, CH)])
            return c

        lax.fori_loop(0, nsteps, body, 0)

    return k(P, Q, row, col)


def _sc_degree(col, w):
    CH = 2000
    nsteps = EPW // CH

    @functools.partial(
        pl.kernel,
        out_type=jax.ShapeDtypeStruct((2, N), jnp.float32),
        mesh=plsc.VectorSubcoreMesh(**_MESH),
        scratch_types=[
            pltpu.VMEM((CH,), jnp.int32),
            pltpu.VMEM((CH,), jnp.float32),
            pltpu.VMEM_SHARED((N,), jnp.float32),
        ],
    )
    def k(col_hbm, w_hbm, out_hbm, idx_v, w_v, acc_sh):
        cid = lax.axis_index("c")
        sid = lax.axis_index("s")

        def z(i, c):
            w_v[pl.ds(i * 16, 16)] = jnp.zeros((16,), jnp.float32)
            return c

        lax.fori_loop(0, CH // 16, z, 0)

        nz = N // CH
        @pl.when(sid < nz)
        def _():
            pltpu.sync_copy(w_v, acc_sh.at[pl.ds(sid * CH, CH)])

        plsc.subcore_barrier()
        base = _wid() * EPW

        def body(i, c):
            off = base + i * CH
            pltpu.sync_copy(col_hbm.at[pl.ds(off, CH)], idx_v)
            pltpu.sync_copy(w_hbm.at[pl.ds(off, CH)], w_v)
            pltpu.sync_copy(w_v, acc_sh.at[idx_v], add=True)
            return c

        lax.fori_loop(0, nsteps, body, 0)
        plsc.subcore_barrier()

        @pl.when(sid == 0)
        def _():
            pltpu.sync_copy(acc_sh, out_hbm.at[cid])

    return k(col, w)


def _sc_aggregate(y, row, col, w):
    CH = 200
    nsteps = EPW // CH
    nz = N // CH

    @functools.partial(
        pl.kernel,
        out_type=jax.ShapeDtypeStruct((2, N, D), jnp.float32),
        mesh=plsc.VectorSubcoreMesh(**_MESH),
        scratch_types=[
            pltpu.VMEM((CH,), jnp.int32),
            pltpu.VMEM((CH + 16,), jnp.float32),
            pltpu.VMEM((CH, D), jnp.float32),
            pltpu.VMEM_SHARED((N, D), jnp.float32),
            pltpu.SemaphoreType.DMA,
        ],
    )
    def k(y_hbm, row_hbm, col_hbm, w_hbm, out_hbm, idx_v, w_v, buf, acc_sh, sem):
        cid = lax.axis_index("c")
        sid = lax.axis_index("s")

        def zrow(i, c):
            for kk in range(D // 16):
                buf[i, pl.ds(kk * 16, 16)] = jnp.zeros((16,), jnp.float32)
            return c

        lax.fori_loop(0, CH, zrow, 0)

        for jj in range((nz + 15) // 16):
            j = sid + 16 * jj

            @pl.when(j < nz)
            def _():
                pltpu.sync_copy(buf, acc_sh.at[pl.ds(j * CH, CH)])

        plsc.subcore_barrier()
        base = _wid() * EPW

        def body(i, c):
            off = base + i * CH
            pltpu.sync_copy(row_hbm.at[pl.ds(off, CH)], idx_v)
            pltpu.sync_copy(w_hbm.at[pl.ds(off, CH)], w_v.at[pl.ds(0, CH)])
            pltpu.async_copy(y_hbm.at[idx_v], buf, sem).wait()

            def srow(e, c2):
                we = w_v[pl.ds(e, 16)][0]
                for kk in range(D // 16):
                    buf[e, pl.ds(kk * 16, 16)] = buf[e, pl.ds(kk * 16, 16)] * we
                return c2

            lax.fori_loop(0, CH, srow, 0)
            pltpu.sync_copy(col_hbm.at[pl.ds(off, CH)], idx_v)
            pltpu.sync_copy(buf, acc_sh.at[idx_v], add=True)
            return c

        lax.fori_loop(0, nsteps, body, 0)
        plsc.subcore_barrier()

        for jj in range((nz + 15) // 16):
            j = sid + 16 * jj

            @pl.when(j < nz)
            def _():
                pltpu.sync_copy(acc_sh.at[pl.ds(j * CH, CH)],
                                out_hbm.at[cid, pl.ds(j * CH, CH)])

    return k(y, row, col, w)


_NB = 2000


def _tc_node1(x, W1a, W1b, b1):
    def body(x_ref, wa_ref, wb_ref, b1_ref, p_ref, q_ref):
        xb = x_ref[...]
        p_ref[...] = jnp.dot(xb, wa_ref[...], preferred_element_type=jnp.float32)
        q_ref[...] = jnp.dot(xb, wb_ref[...], preferred_element_type=jnp.float32) + b1_ref[...]

    return pl.pallas_call(
        body,
        grid=(N // _NB,),
        in_specs=[
            pl.BlockSpec((_NB, D), lambda i: (i, 0)),
            pl.BlockSpec((D, ED), lambda i: (0, 0)),
            pl.BlockSpec((D, ED), lambda i: (0, 0)),
            pl.BlockSpec((1, ED), lambda i: (0, 0)),
        ],
        out_specs=[
            pl.BlockSpec((_NB, ED), lambda i: (i, 0)),
            pl.BlockSpec((_NB, ED), lambda i: (i, 0)),
        ],
        out_shape=[jax.ShapeDtypeStruct((N, ED), jnp.float32)] * 2,
    )(x, W1a, W1b, b1.reshape(1, ED))


_EB = 4000
_NEB = E // _EB


_EB2 = _EB // 2


def _tc_edge1(PQ2, ei_W2, ei_b2, ew_W, ew_b):
    W2d = jnp.zeros((D, D), jnp.float32)
    W2d = W2d.at[:ED, :ED].set(ei_W2).at[ED:, ED:].set(ei_W2)
    b2p = jnp.concatenate([ei_b2, ei_b2]).reshape(1, D)
    ewp = jnp.concatenate([ew_W[:, 0], ew_W[:, 0]]).reshape(1, D)

    Mb = jnp.zeros((8, D), jnp.float32)
    Mb = Mb.at[0, :ED].set(ew_W[:, 0]).at[1, ED:].set(ew_W[:, 0])

    def body(pq_ref, w2d_ref, b2p_ref, mb_ref, ewb_ref, ef_ref, we_ref, wo_ref):
        efp = jnp.dot(jnp.maximum(pq_ref[...], 0.0), w2d_ref[...],
                      preferred_element_type=jnp.float32) + b2p_ref[...]
        ef_ref[...] = efp
        l = lax.dot_general(mb_ref[...], efp, (((1,), (1,)), ((), ())),
                            preferred_element_type=jnp.float32)
        z = jax.nn.sigmoid(l[0:2] + ewb_ref[0, 0])
        we_ref[...] = z[0:1].reshape(1, 1, _EB2)
        wo_ref[...] = z[1:2].reshape(1, 1, _EB2)

    ef, we, wo = pl.pallas_call(
        body,
        grid=(_NEB,),
        in_specs=[
            pl.BlockSpec((_EB2, D), lambda i: (i, 0)),
            pl.BlockSpec((D, D), lambda i: (0, 0)),
            pl.BlockSpec((1, D), lambda i: (0, 0)),
            pl.BlockSpec((8, D), lambda i: (0, 0)),
            pl.BlockSpec((1, 1), lambda i: (0, 0)),
        ],
        out_specs=[
            pl.BlockSpec((_EB2, D), lambda i: (i, 0)),
            pl.BlockSpec((1, 1, _EB2), lambda i: (i, 0, 0)),
            pl.BlockSpec((1, 1, _EB2), lambda i: (i, 0, 0)),
        ],
        out_shape=[
            jax.ShapeDtypeStruct((E // 2, D), jnp.float32),
            jax.ShapeDtypeStruct((_NEB, 1, _EB2), jnp.float32),
            jax.ShapeDtypeStruct((_NEB, 1, _EB2), jnp.float32),
        ],
    )(PQ2, W2d, b2p, Mb, ew_b.reshape(1, 1))
    w1 = jnp.concatenate([we.reshape(E // 2), wo.reshape(E // 2)])
    return ef, w1


def _tc_mid(degp, x, gW):
    def body(degp_ref, x_ref, gw_ref, dinv_ref, y_ref):
        deg = 1.0 + degp_ref[0] + degp_ref[1]
        dinv = lax.rsqrt(deg)
        dinv_ref[...] = dinv
        xw = jnp.dot(x_ref[...], gw_ref[...], preferred_element_type=jnp.float32)
        y_ref[...] = xw * dinv

    return pl.pallas_call(
        body,
        grid=(N // _NB,),
        in_specs=[
            pl.BlockSpec((2, _NB, 1), lambda i: (0, i, 0)),
            pl.BlockSpec((_NB, D), lambda i: (i, 0)),
            pl.BlockSpec((D, D), lambda i: (0, 0)),
        ],
        out_specs=[
            pl.BlockSpec((_NB, 1), lambda i: (i, 0)),
            pl.BlockSpec((_NB, D), lambda i: (i, 0)),
        ],
        out_shape=[
            jax.ShapeDtypeStruct((N, 1), jnp.float32),
            jax.ShapeDtypeStruct((N, D), jnp.float32),
        ],
    )(degp.reshape(2, N, 1), x, gW)


def _gelu_ln(pre, g_ref, b_ref):
    h = 0.5 * pre * (1.0 + lax.erf(pre * (2.0 ** -0.5)))
    m = jnp.mean(h, axis=-1, keepdims=True)
    v = jnp.mean((h - m) ** 2, axis=-1, keepdims=True)
    return (h - m) * lax.rsqrt(v + 1e-5) * g_ref[...] + b_ref[...]


def _tc_out1(accp, y1, dinv1, gcn1_b, n1_g, n1_b, eu_A, eu_B, eu_b1, gcn2_W):
    def body(accp_ref, y_ref, dinv_ref, gb_ref, g_ref, b_ref, a_ref, bb_ref,
             eb1_ref, gw2_ref, r_ref, s_ref, xw2_ref):
        pre = dinv_ref[...] * (accp_ref[0] + accp_ref[1] + y_ref[...]) + gb_ref[...]
        out1 = _gelu_ln(pre, g_ref, b_ref)
        r_ref[...] = jnp.dot(out1, a_ref[...], preferred_element_type=jnp.float32)
        s_ref[...] = jnp.dot(out1, bb_ref[...], preferred_element_type=jnp.float32) + eb1_ref[...]
        xw2_ref[...] = jnp.dot(out1, gw2_ref[...], preferred_element_type=jnp.float32)

    return pl.pallas_call(
        body,
        grid=(N // _NB,),
        in_specs=[
            pl.BlockSpec((2, _NB, D), lambda i: (0, i, 0)),
            pl.BlockSpec((_NB, D), lambda i: (i, 0)),
            pl.BlockSpec((_NB, 1), lambda i: (i, 0)),
            pl.BlockSpec((1, D), lambda i: (0, 0)),
            pl.BlockSpec((1, D), lambda i: (0, 0)),
            pl.BlockSpec((1, D), lambda i: (0, 0)),
            pl.BlockSpec((D, D), lambda i: (0, 0)),
            pl.BlockSpec((D, D), lambda i: (0, 0)),
            pl.BlockSpec((1, D), lambda i: (0, 0)),
            pl.BlockSpec((D, D), lambda i: (0, 0)),
        ],
        out_specs=[
            pl.BlockSpec((_NB, D), lambda i: (i, 0)),
            pl.BlockSpec((_NB, D), lambda i: (i, 0)),
            pl.BlockSpec((_NB, D), lambda i: (i, 0)),
        ],
        out_shape=[jax.ShapeDtypeStruct((N, D), jnp.float32)] * 3,
    )(accp, y1, dinv1, gcn1_b.reshape(1, D), n1_g.reshape(1, D),
      n1_b.reshape(1, D), eu_A, eu_B, eu_b1.reshape(1, D), gcn2_W)


def _tc_edge2(RS, ef, eu_C, eu_W2, eu_b2, ne_g, ne_b, ew_W, ew_b):
    gw = ne_g * ew_W[:, 0]
    cb = (jnp.sum(ne_b * ew_W[:, 0]) + ew_b[0]).reshape(1, 1)
    noff = E // 2 // _EB2
    Me = jnp.zeros((8, ED), jnp.float32)
    Me = Me.at[0].set(1.0).at[1].set(gw - jnp.sum(gw) / ED)
    Ms = jnp.zeros((8, ED), jnp.float32).at[0].set(1.0)

    def _half(rs, ef, c_ref, w2_ref, b2_ref, me_ref, ms_ref, cb_ref):
        h = jnp.maximum(
            rs + jnp.dot(ef, c_ref[...], preferred_element_type=jnp.float32), 0.0)
        upd = jnp.dot(h, w2_ref[...], preferred_element_type=jnp.float32) + b2_ref[...]
        ef2 = ef + upd
        dims = (((1,), (1,)), ((), ()))
        t1 = lax.dot_general(me_ref[...], ef2, dims,
                             preferred_element_type=jnp.float32)
        t2 = lax.dot_general(ms_ref[...], ef2 * ef2, dims,
                             preferred_element_type=jnp.float32)
        s0, s1, s2 = t1[0:1], t1[1:2], t2[0:1]
        m = s0 * (1.0 / ED)
        var = s2 * (1.0 / ED) - m * m
        logit = s1 * lax.rsqrt(var + 1e-5) + cb_ref[0, 0]
        return jax.nn.sigmoid(logit).reshape(1, 1, _EB2)

    def body(rs1_ref, rs2_ref, ef_ref, c_ref, w2_ref, b2_ref, me_ref, ms_ref,
             cb_ref, w2e_ref, w2o_ref):
        efp = ef_ref[...]
        w2e_ref[...] = _half(rs1_ref[...], efp[:, :ED], c_ref, w2_ref, b2_ref,
                             me_ref, ms_ref, cb_ref)
        w2o_ref[...] = _half(rs2_ref[...], efp[:, ED:], c_ref, w2_ref, b2_ref,
                             me_ref, ms_ref, cb_ref)

    w2e, w2o = pl.pallas_call(
        body,
        grid=(_NEB,),
        in_specs=[
            pl.BlockSpec((_EB2, D), lambda i: (i, 0)),
            pl.BlockSpec((_EB2, D), lambda i: (i + noff, 0)),
            pl.BlockSpec((_EB2, D), lambda i: (i, 0)),
            pl.BlockSpec((ED, D), lambda i: (0, 0)),
            pl.BlockSpec((D, ED), lambda i: (0, 0)),
            pl.BlockSpec((1, ED), lambda i: (0, 0)),
            pl.BlockSpec((8, ED), lambda i: (0, 0)),
            pl.BlockSpec((8, ED), lambda i: (0, 0)),
            pl.BlockSpec((1, 1), lambda i: (0, 0)),
        ],
        out_specs=[
            pl.BlockSpec((1, 1, _EB2), lambda i: (i, 0, 0)),
            pl.BlockSpec((1, 1, _EB2), lambda i: (i, 0, 0)),
        ],
        out_shape=[
            jax.ShapeDtypeStruct((_NEB, 1, _EB2), jnp.float32),
            jax.ShapeDtypeStruct((_NEB, 1, _EB2), jnp.float32),
        ],
    )(RS, RS, ef, eu_C, eu_W2, eu_b2.reshape(1, ED), Me, Ms, cb)
    return jnp.concatenate([w2e.reshape(E // 2), w2o.reshape(E // 2)])


def _tc_out2(accp, y2, dinv2, gcn2_b, n2_g, n2_b):
    def body(accp_ref, y_ref, dinv_ref, gb_ref, g_ref, b_ref, o_ref):
        pre = dinv_ref[...] * (accp_ref[0] + accp_ref[1] + y_ref[...]) + gb_ref[...]
        o_ref[...] = _gelu_ln(pre, g_ref, b_ref)

    return pl.pallas_call(
        body,
        grid=(N // _NB,),
        in_specs=[
            pl.BlockSpec((2, _NB, D), lambda i: (0, i, 0)),
            pl.BlockSpec((_NB, D), lambda i: (i, 0)),
            pl.BlockSpec((_NB, 1), lambda i: (i, 0)),
            pl.BlockSpec((1, D), lambda i: (0, 0)),
            pl.BlockSpec((1, D), lambda i: (0, 0)),
            pl.BlockSpec((1, D), lambda i: (0, 0)),
        ],
        out_specs=pl.BlockSpec((_NB, D), lambda i: (i, 0)),
        out_shape=jax.ShapeDtypeStruct((N, D), jnp.float32),
    )(accp, y2, dinv2, gcn2_b.reshape(1, D), n2_g.reshape(1, D),
      n2_b.reshape(1, D))


def kernel(x, edge_index, gcn1_W, gcn1_b, gcn2_W, gcn2_b, n1_g, n1_b, n2_g,
           n2_b, ne_g, ne_b, ei_W1, ei_b1, ei_W2, ei_b2, eu_W1, eu_b1, eu_W2,
           eu_b2, ew_W, ew_b):
    row = edge_index[0]
    col = edge_index[1]

    P, Q = _tc_node1(x, ei_W1[:D], ei_W1[D:], ei_b1)
    PQ2 = _sc_gather2(P, Q, row, col, ED, 1000, pair_out=True)
    ef, w1 = _tc_edge1(PQ2, ei_W2, ei_b2, ew_W, ew_b)

    deg1p = _sc_degree(col, w1)
    dinv1, y1 = _tc_mid(deg1p, x, gcn1_W)
    acc1p = _sc_aggregate(y1, row, col, w1)
    R, S, xw2 = _tc_out1(acc1p, y1, dinv1, gcn1_b, n1_g, n1_b,
                         eu_W1[:D], eu_W1[D:2 * D], eu_b1, gcn2_W)

    RS = _sc_gather2(R, S, row, col, D, 400)
    w2 = _tc_edge2(RS, ef, eu_W1[2 * D:], eu_W2, eu_b2, ne_g, ne_b, ew_W, ew_b)

    deg2p = _sc_degree(col, w2)

    def _scale_body(degp_ref, xw_ref, dinv_ref, y_ref):
        deg = 1.0 + degp_ref[0] + degp_ref[1]
        dinv = lax.rsqrt(deg)
        dinv_ref[...] = dinv
        y_ref[...] = xw_ref[...] * dinv

    dinv2, y2 = pl.pallas_call(
        _scale_body,
        grid=(N // _NB,),
        in_specs=[
            pl.BlockSpec((2, _NB, 1), lambda i: (0, i, 0)),
            pl.BlockSpec((_NB, D), lambda i: (i, 0)),
        ],
        out_specs=[
            pl.BlockSpec((_NB, 1), lambda i: (i, 0)),
            pl.BlockSpec((_NB, D), lambda i: (i, 0)),
        ],
        out_shape=[
            jax.ShapeDtypeStruct((N, 1), jnp.float32),
            jax.ShapeDtypeStruct((N, D), jnp.float32),
        ],
    )(deg2p.reshape(2, N, 1), xw2)

    acc2p = _sc_aggregate(y2, row, col, w2)
    return _tc_out2(acc2p, y2, dinv2, gcn2_b, n2_g, n2_b)
